# baseline (speedup 1.0000x reference)
import functools

import jax
import jax.numpy as jnp
from jax import lax
from jax.experimental import pallas as pl
from jax.experimental.pallas import tpu as pltpu
from jax.experimental.pallas import tpu_sc as plsc

N = 10000
E = 320000
IN_DIM = 128
HID_DIM = 256
OUT_DIM = 128
K = 10

T = 16
D = OUT_DIM
EC = 128
EBR = 2560
TBLK = EBR // T
BB = 8
NBLK = TBLK // BB
RB = 40
NCHN = N // RB
NIT = -(-NCHN // T)
NPAD = 8


def _mlp_body(x_ref, w1_ref, b1_ref, w2_ref, b2_ref, o_ref):
    h = jnp.maximum(
        jnp.dot(x_ref[...], w1_ref[...], preferred_element_type=jnp.float32)
        + b1_ref[...][None, :], 0.0)
    o_ref[...] = (
        jnp.dot(h, w2_ref[...], preferred_element_type=jnp.float32)
        + b2_ref[...][None, :])


def _mlp(x, W1, b1, W2, b2):
    BR = 1000
    return pl.pallas_call(
        _mlp_body,
        grid=(N // BR,),
        in_specs=[
            pl.BlockSpec((BR, IN_DIM), lambda i: (i, 0)),
            pl.BlockSpec((IN_DIM, HID_DIM), lambda i: (0, 0)),
            pl.BlockSpec((HID_DIM,), lambda i: (0,)),
            pl.BlockSpec((HID_DIM, OUT_DIM), lambda i: (0, 0)),
            pl.BlockSpec((OUT_DIM,), lambda i: (0,)),
        ],
        out_specs=pl.BlockSpec((BR, OUT_DIM), lambda i: (i, 0)),
        out_shape=jax.ShapeDtypeStruct((N, OUT_DIM), jnp.float32),
    )(x, W1, b1, W2, b2)


def _rsqrt16(v):
    i = lax.bitcast_convert_type(v, jnp.int32)
    i = jnp.int32(0x5F3759DF) - lax.shift_right_arithmetic(i, jnp.int32(1))
    y = lax.bitcast_convert_type(i, jnp.float32)
    vh = v * jnp.float32(-0.5)
    for _ in range(4):
        y = y * (jnp.float32(1.5) + vh * y * y)
    return jnp.where(v > 0.0, y, jnp.float32(0.0))


def _mesh():
    return plsc.VectorSubcoreMesh(
        core_axis_name="c", subcore_axis_name="s", num_cores=1)


def _node_loop(tid, body):
    def outer(i, _):
        cid = i * T + tid

        @pl.when(cid < NCHN)
        def _():
            body(pl.multiple_of(cid * RB, RB))
        return 0
    lax.fori_loop(0, NIT, outer, 0)


def _fill_rows(buf, r0, nrows, vec16):
    def frow(r, _):
        for c in range(D // 16):
            buf[r0 + r, pl.ds(c * 16, 16)] = vec16
        return 0
    lax.fori_loop(0, nrows, frow, 0)


def _degnorm_body(srcM_hbm, dstM_hbm,
                  ns_hbm, nd_hbm,
                  hist_sh,
                  sidx_v, ones_v, acc_v, zbuf_v):
    tid = lax.axis_index("s")

    _fill_rows(ones_v, 0, EC, jnp.ones((16,), jnp.float32))
    _fill_rows(zbuf_v, 0, RB, jnp.zeros((16,), jnp.float32))

    def zero_hist(base):
        pltpu.sync_copy(zbuf_v, hist_sh.at[pl.ds(base, RB)])
    _node_loop(tid, zero_hist)

    @pl.when(tid == 0)
    def _():
        pltpu.sync_copy(zbuf_v.at[pl.ds(0, NPAD)],
                        hist_sh.at[pl.ds(N, NPAD)])
    plsc.subcore_barrier()

    def round_(edges_hbm, out_hbm, last):
        def blk(b, _):
            row0 = pl.multiple_of(tid * TBLK + b * BB, BB)
            pltpu.sync_copy(edges_hbm.at[pl.ds(row0, BB)], sidx_v)
            for r in range(BB):
                pltpu.sync_copy(ones_v, hist_sh.at[sidx_v.at[r]], add=True)
            return 0
        lax.fori_loop(0, NBLK, blk, 0)
        plsc.subcore_barrier()

        def norm_body(base):
            pltpu.sync_copy(hist_sh.at[pl.ds(base, RB)], acc_v)
            if not last:
                pltpu.sync_copy(zbuf_v, hist_sh.at[pl.ds(base, RB)])

            def nrow(r, _):
                for c in range(D // 16):
                    sl = pl.ds(c * 16, 16)
                    acc_v[r, sl] = _rsqrt16(acc_v[r, sl])
                return 0
            lax.fori_loop(0, RB, nrow, 0)
            pltpu.sync_copy(acc_v, out_hbm.at[pl.ds(base, RB)])
        _node_loop(tid, norm_body)
        plsc.subcore_barrier()

    round_(srcM_hbm, ns_hbm, False)
    round_(dstM_hbm, nd_hbm, True)


def _sc_degnorm(srcM, dstM):
    f = functools.partial(
        pl.kernel,
        out_type=[
            jax.ShapeDtypeStruct((N, D), jnp.float32),
            jax.ShapeDtypeStruct((N, D), jnp.float32),
        ],
        mesh=_mesh(),
        scratch_types=[
            pltpu.VMEM_SHARED((N + NPAD, D), jnp.float32),
            pltpu.VMEM((BB, EC), jnp.int32),
            pltpu.VMEM((EC, D), jnp.float32),
            pltpu.VMEM((RB, D), jnp.float32),
            pltpu.VMEM((RB, D), jnp.float32),
        ],
    )
    return f(_degnorm_body)(srcM, dstM)


def _hops_body(h0_hbm, srcM_hbm, dstM_hbm, ns_hbm, nd_hbm,
               hout_hbm, g_hbm,
               agg_sh,
               sidx_v, didx_v, rows0_v, rows1_v, acc_v,
               sem0, sem1, sem2, sem3):
    tid = lax.axis_index("s")
    rows = (rows0_v, rows1_v)
    sems = (sem0, sem1, sem2, sem3)
    zeros16 = jnp.zeros((16,), jnp.float32)

    _fill_rows(rows0_v, 80, RB, zeros16)

    def g0_body(base):
        pltpu.sync_copy(ns_hbm.at[pl.ds(base, RB)], rows1_v.at[pl.ds(0, RB)])
        pltpu.sync_copy(h0_hbm.at[pl.ds(base, RB)], acc_v)

        def srow(r, _):
            for c in range(D // 16):
                sl = pl.ds(c * 16, 16)
                acc_v[r, sl] = acc_v[r, sl] * rows1_v[r, sl]
            return 0
        lax.fori_loop(0, RB, srow, 0)
        pltpu.sync_copy(acc_v, g_hbm.at[pl.ds(base, RB)])
        pltpu.sync_copy(rows0_v.at[pl.ds(80, RB)], agg_sh.at[pl.ds(base, RB)])
    _node_loop(tid, g0_body)

    @pl.when(tid == 0)
    def _():
        pltpu.sync_copy(rows0_v.at[pl.ds(80, NPAD)], g_hbm.at[pl.ds(N, NPAD)])
        pltpu.sync_copy(rows0_v.at[pl.ds(80, NPAD)],
                        agg_sh.at[pl.ds(N, NPAD)])
    plsc.subcore_barrier()

    def hop(k, _):
        def blk(b, _):
            row0 = pl.multiple_of(tid * TBLK + b * BB, BB)
            pltpu.sync_copy(srcM_hbm.at[pl.ds(row0, BB)], sidx_v)
            pltpu.sync_copy(dstM_hbm.at[pl.ds(row0, BB)], didx_v)
            def fire(r):
                buf = rows[r % 2]
                return [
                    pltpu.async_copy(
                        g_hbm.at[sidx_v.at[r, pl.ds(h * 64, 64)]],
                        buf.at[pl.ds(h * 64, 64)],
                        sems[(r % 2) * 2 + h])
                    for h in range(2)
                ]

            cps = [None, None]
            cps[0] = fire(0)
            for r in range(BB):
                if r + 1 < BB:
                    cps[(r + 1) % 2] = fire(r + 1)
                for cp in cps[r % 2]:
                    cp.wait()
                pltpu.sync_copy(rows[r % 2], agg_sh.at[didx_v.at[r]],
                                add=True)
            return 0
        lax.fori_loop(0, NBLK, blk, 0)
        plsc.subcore_barrier()

        _fill_rows(rows0_v, 80, RB, zeros16)

        def wb_body(base):
            pltpu.sync_copy(agg_sh.at[pl.ds(base, RB)], acc_v)
            pltpu.sync_copy(nd_hbm.at[pl.ds(base, RB)],
                            rows1_v.at[pl.ds(0, RB)])

            def srow(r, _):
                for c in range(D // 16):
                    sl = pl.ds(c * 16, 16)
                    acc_v[r, sl] = acc_v[r, sl] * rows1_v[r, sl]
                return 0
            lax.fori_loop(0, RB, srow, 0)
            hrow = pl.multiple_of(k * N + base, 8)
            pltpu.sync_copy(acc_v, hout_hbm.at[pl.ds(hrow, RB)])

            @pl.when(k < K - 1)
            def _():
                pltpu.sync_copy(ns_hbm.at[pl.ds(base, RB)],
                                rows1_v.at[pl.ds(0, RB)])

                def srow2(r, _):
                    for c in range(D // 16):
                        sl = pl.ds(c * 16, 16)
                        acc_v[r, sl] = acc_v[r, sl] * rows1_v[r, sl]
                    return 0
                lax.fori_loop(0, RB, srow2, 0)
                pltpu.sync_copy(acc_v, g_hbm.at[pl.ds(base, RB)])
                pltpu.sync_copy(rows0_v.at[pl.ds(80, RB)],
                                agg_sh.at[pl.ds(base, RB)])
        _node_loop(tid, wb_body)
        plsc.subcore_barrier()
        return 0
    lax.fori_loop(0, K, hop, 0)


def _sc_hops(h0, srcM, dstM, ns, nd):
    f = functools.partial(
        pl.kernel,
        out_type=[
            jax.ShapeDtypeStruct((K * N, D), jnp.float32),
            jax.ShapeDtypeStruct((N + NPAD, D), jnp.float32),
        ],
        mesh=_mesh(),
        scratch_types=[
            pltpu.VMEM_SHARED((N + NPAD, D), jnp.float32),
            pltpu.VMEM((BB, EC), jnp.int32),
            pltpu.VMEM((BB, EC), jnp.int32),
            pltpu.VMEM((EC, D), jnp.float32),
            pltpu.VMEM((EC, D), jnp.float32),
            pltpu.VMEM((RB, D), jnp.float32),
            pltpu.SemaphoreType.DMA,
            pltpu.SemaphoreType.DMA,
            pltpu.SemaphoreType.DMA,
            pltpu.SemaphoreType.DMA,
        ],
    )
    return f(_hops_body)(h0, srcM, dstM, ns, nd)


def _gate_body(h0_ref, hh_ref, pw_ref, pb_ref, o_ref):
    pw = pw_ref[...]
    pb = pb_ref[...]
    h0 = h0_ref[...]
    s = jax.nn.sigmoid(jnp.dot(h0, pw, preferred_element_type=jnp.float32) + pb)
    acc = s * h0
    for k in range(K):
        hk = hh_ref[k]
        sk = jax.nn.sigmoid(
            jnp.dot(hk, pw, preferred_element_type=jnp.float32) + pb)
        acc = acc + sk * hk
    o_ref[...] = acc


def _gating(h0, hh, proj_w, proj_b):
    BR = 1000
    return pl.pallas_call(
        _gate_body,
        grid=(N // BR,),
        in_specs=[
            pl.BlockSpec((BR, D), lambda i: (i, 0)),
            pl.BlockSpec((K, BR, D), lambda i: (0, i, 0)),
            pl.BlockSpec((D, 1), lambda i: (0, 0)),
            pl.BlockSpec((1,), lambda i: (0,)),
        ],
        out_specs=pl.BlockSpec((BR, D), lambda i: (i, 0)),
        out_shape=jax.ShapeDtypeStruct((N, D), jnp.float32),
    )(h0, hh, proj_w, proj_b)


@jax.jit
def kernel(x, edge_index, W1, b1, W2, b2, proj_w, proj_b):
    h0 = _mlp(x, W1, b1, W2, b2)
    pad = jnp.full((EBR * EC - E,), N, jnp.int32)
    srcM = jnp.concatenate([edge_index[0], pad]).reshape(EBR, EC)
    dstM = jnp.concatenate([edge_index[1], pad]).reshape(EBR, EC)
    ns, nd = _sc_degnorm(srcM, dstM)
    hh, _g = _sc_hops(h0, srcM, dstM, ns, nd)
    return _gating(h0, hh.reshape(K, N, D), proj_w, proj_b)


# speedup vs baseline: 1.2878x; 1.2878x over previous
import functools

import jax
import jax.numpy as jnp
from jax import lax
from jax.experimental import pallas as pl
from jax.experimental.pallas import tpu as pltpu
from jax.experimental.pallas import tpu_sc as plsc

N = 10000
E = 320000
IN_DIM = 128
HID_DIM = 256
OUT_DIM = 128
K = 10

T = 16
D = OUT_DIM
EC = 128
EBR = 2560
TBLK = EBR // T
BB = 8
NBLK = TBLK // BB
RB = 40
NCHN = N // RB
NIT = -(-NCHN // T)
NPAD = 8


def _mlp_body(x_ref, w1_ref, b1_ref, w2_ref, b2_ref, o_ref):
    h = jnp.maximum(
        jnp.dot(x_ref[...], w1_ref[...], preferred_element_type=jnp.float32)
        + b1_ref[...][None, :], 0.0)
    o_ref[...] = (
        jnp.dot(h, w2_ref[...], preferred_element_type=jnp.float32)
        + b2_ref[...][None, :])


def _mlp(x, W1, b1, W2, b2):
    BR = 1000
    return pl.pallas_call(
        _mlp_body,
        grid=(N // BR,),
        in_specs=[
            pl.BlockSpec((BR, IN_DIM), lambda i: (i, 0)),
            pl.BlockSpec((IN_DIM, HID_DIM), lambda i: (0, 0)),
            pl.BlockSpec((HID_DIM,), lambda i: (0,)),
            pl.BlockSpec((HID_DIM, OUT_DIM), lambda i: (0, 0)),
            pl.BlockSpec((OUT_DIM,), lambda i: (0,)),
        ],
        out_specs=pl.BlockSpec((BR, OUT_DIM), lambda i: (i, 0)),
        out_shape=jax.ShapeDtypeStruct((N, OUT_DIM), jnp.float32),
    )(x, W1, b1, W2, b2)


def _rsqrt16(v):
    i = lax.bitcast_convert_type(v, jnp.int32)
    i = jnp.int32(0x5F3759DF) - lax.shift_right_arithmetic(i, jnp.int32(1))
    y = lax.bitcast_convert_type(i, jnp.float32)
    vh = v * jnp.float32(-0.5)
    for _ in range(4):
        y = y * (jnp.float32(1.5) + vh * y * y)
    return jnp.where(v > 0.0, y, jnp.float32(0.0))


def _mesh(num_cores):
    return plsc.VectorSubcoreMesh(
        core_axis_name="c", subcore_axis_name="s", num_cores=num_cores)


def _node_loop(tid, body):
    def outer(i, _):
        cid = i * T + tid

        @pl.when(cid < NCHN)
        def _():
            body(pl.multiple_of(cid * RB, RB))
        return 0
    lax.fori_loop(0, NIT, outer, 0)


def _fill_rows(buf, r0, nrows, vec16):
    def frow(r, _):
        for c in range(D // 16):
            buf[r0 + r, pl.ds(c * 16, 16)] = vec16
        return 0
    lax.fori_loop(0, nrows, frow, 0)


def _degnorm_body(srcM_hbm, dstM_hbm,
                  ns_hbm, nd_hbm, nc_hbm,
                  hist_sh,
                  sidx_v, ones_v, acc_v, zbuf_v):
    tid = lax.axis_index("s")

    _fill_rows(ones_v, 0, EC, jnp.ones((16,), jnp.float32))
    _fill_rows(zbuf_v, 0, RB, jnp.zeros((16,), jnp.float32))

    def zero_hist(base):
        pltpu.sync_copy(zbuf_v, hist_sh.at[pl.ds(base, RB)])
    _node_loop(tid, zero_hist)

    @pl.when(tid == 0)
    def _():
        pltpu.sync_copy(zbuf_v.at[pl.ds(0, NPAD)],
                        hist_sh.at[pl.ds(N, NPAD)])
    plsc.subcore_barrier()

    def round_(edges_hbm, out_hbm, last):
        def blk(b, _):
            row0 = pl.multiple_of(tid * TBLK + b * BB, BB)
            pltpu.sync_copy(edges_hbm.at[pl.ds(row0, BB)], sidx_v)
            for r in range(BB):
                pltpu.sync_copy(ones_v, hist_sh.at[sidx_v.at[r]], add=True)
            return 0
        lax.fori_loop(0, NBLK, blk, 0)
        plsc.subcore_barrier()

        def norm_body(base):
            pltpu.sync_copy(hist_sh.at[pl.ds(base, RB)], acc_v)
            if not last:
                pltpu.sync_copy(zbuf_v, hist_sh.at[pl.ds(base, RB)])

            def nrow(r, _):
                for c in range(D // 16):
                    sl = pl.ds(c * 16, 16)
                    acc_v[r, sl] = _rsqrt16(acc_v[r, sl])
                return 0
            lax.fori_loop(0, RB, nrow, 0)
            pltpu.sync_copy(acc_v, out_hbm.at[pl.ds(base, RB)])
            if last:
                pltpu.sync_copy(ns_hbm.at[pl.ds(base, RB)], zbuf_v)

                def crow(r, _):
                    for c in range(D // 16):
                        sl = pl.ds(c * 16, 16)
                        acc_v[r, sl] = acc_v[r, sl] * zbuf_v[r, sl]
                    return 0
                lax.fori_loop(0, RB, crow, 0)
                pltpu.sync_copy(acc_v, nc_hbm.at[pl.ds(base, RB)])
        _node_loop(tid, norm_body)
        plsc.subcore_barrier()

    round_(srcM_hbm, ns_hbm, False)
    round_(dstM_hbm, nd_hbm, True)


def _sc_degnorm(srcM, dstM):
    f = functools.partial(
        pl.kernel,
        out_type=[
            jax.ShapeDtypeStruct((N, D), jnp.float32),
            jax.ShapeDtypeStruct((N, D), jnp.float32),
            jax.ShapeDtypeStruct((N, D), jnp.float32),
        ],
        mesh=_mesh(1),
        scratch_types=[
            pltpu.VMEM_SHARED((N + NPAD, D), jnp.float32),
            pltpu.VMEM((BB, EC), jnp.int32),
            pltpu.VMEM((EC, D), jnp.float32),
            pltpu.VMEM((RB, D), jnp.float32),
            pltpu.VMEM((RB, D), jnp.float32),
        ],
    )
    return f(_degnorm_body)(srcM, dstM)


def _hop_body(first, prev_hbm, srcM_hbm, dstM_hbm, norm_hbm,
              pout_hbm,
              agg_sh,
              sidx_v, didx_v, rows0_v, rows1_v, acc_v,
              sem0, sem1):
    tid = lax.axis_index("s")
    cid = lax.axis_index("c")
    rows = (rows0_v, rows1_v)
    sems = (sem0, sem1)
    zeros16 = jnp.zeros((16,), jnp.float32)

    _fill_rows(rows1_v, 0, RB, zeros16)

    def g_body(base):
        pltpu.sync_copy(norm_hbm.at[pl.ds(base, RB)],
                        rows0_v.at[pl.ds(0, RB)])
        if first:
            pltpu.sync_copy(prev_hbm.at[pl.ds(base, RB)], acc_v)
        else:
            pltpu.sync_copy(prev_hbm.at[0, pl.ds(base, RB)], acc_v)
            pltpu.sync_copy(prev_hbm.at[1, pl.ds(base, RB)],
                            rows0_v.at[pl.ds(40, RB)])

        def srow(r, _):
            for c in range(D // 16):
                sl = pl.ds(c * 16, 16)
                v = acc_v[r, sl]
                if not first:
                    v = v + rows0_v[40 + r, sl]
                acc_v[r, sl] = v * rows0_v[r, sl]
            return 0
        lax.fori_loop(0, RB, srow, 0)
        pltpu.sync_copy(acc_v, pout_hbm.at[cid, pl.ds(base, RB)])
        pltpu.sync_copy(rows1_v.at[pl.ds(0, RB)],
                        agg_sh.at[pl.ds(base, RB)])
    _node_loop(tid, g_body)

    @pl.when(tid == 0)
    def _():
        pltpu.sync_copy(rows1_v.at[pl.ds(0, NPAD)],
                        pout_hbm.at[cid, pl.ds(N, NPAD)])
        pltpu.sync_copy(rows1_v.at[pl.ds(0, NPAD)],
                        agg_sh.at[pl.ds(N, NPAD)])
    plsc.subcore_barrier()

    half = TBLK // 2
    gtab = pout_hbm.at[cid]

    def blk(b, _):
        row0 = pl.multiple_of((cid * T + tid) * half + b * BB, BB)
        pltpu.sync_copy(srcM_hbm.at[pl.ds(row0, BB)], sidx_v)
        pltpu.sync_copy(dstM_hbm.at[pl.ds(row0, BB)], didx_v)
        cps = [None, None]
        cps[0] = pltpu.async_copy(gtab.at[sidx_v.at[0]], rows0_v, sem0)
        for r in range(BB):
            if r + 1 < BB:
                cps[(r + 1) % 2] = pltpu.async_copy(
                    gtab.at[sidx_v.at[r + 1]], rows[(r + 1) % 2],
                    sems[(r + 1) % 2])
            cps[r % 2].wait()
            pltpu.sync_copy(rows[r % 2], agg_sh.at[didx_v.at[r]], add=True)
        return 0
    lax.fori_loop(0, half // BB, blk, 0)
    plsc.subcore_barrier()

    def wb_body(base):
        pltpu.sync_copy(agg_sh.at[pl.ds(base, RB)], acc_v)
        pltpu.sync_copy(acc_v, pout_hbm.at[cid, pl.ds(base, RB)])
    _node_loop(tid, wb_body)


def _hop_call(first):
    return functools.partial(
        pl.kernel,
        out_type=[jax.ShapeDtypeStruct((2, N + NPAD, D), jnp.float32)],
        mesh=_mesh(2),
        scratch_types=[
            pltpu.VMEM_SHARED((N + NPAD, D), jnp.float32),
            pltpu.VMEM((BB, EC), jnp.int32),
            pltpu.VMEM((BB, EC), jnp.int32),
            pltpu.VMEM((EC, D), jnp.float32),
            pltpu.VMEM((EC, D), jnp.float32),
            pltpu.VMEM((RB, D), jnp.float32),
            pltpu.SemaphoreType.DMA,
            pltpu.SemaphoreType.DMA,
        ],
    )(functools.partial(_hop_body, first))


def _sc_hop0(h0, srcM, dstM, ns):
    (p,) = _hop_call(True)(h0, srcM, dstM, ns)
    return p


def _sc_hop(p_prev, srcM, dstM, nc):
    (p,) = _hop_call(False)(p_prev, srcM, dstM, nc)
    return p


def _gate_body(*refs):
    h0_ref = refs[0]
    p_refs = refs[1:1 + K]
    nd_ref, pw_ref, pb_ref, o_ref = refs[1 + K:]
    pw = pw_ref[...]
    pb = pb_ref[...]
    nd = nd_ref[...]
    h0 = h0_ref[...]
    s = jax.nn.sigmoid(jnp.dot(h0, pw, preferred_element_type=jnp.float32) + pb)
    acc = s * h0
    for k in range(K):
        pk = p_refs[k]
        hk = nd * (pk[0] + pk[1])
        sk = jax.nn.sigmoid(
            jnp.dot(hk, pw, preferred_element_type=jnp.float32) + pb)
        acc = acc + sk * hk
    o_ref[...] = acc


def _gating(h0, ps, nd, proj_w, proj_b):
    BR = 1000
    return pl.pallas_call(
        _gate_body,
        grid=(N // BR,),
        in_specs=[pl.BlockSpec((BR, D), lambda i: (i, 0))]
        + [pl.BlockSpec((2, BR, D), lambda i: (0, i, 0)) for _ in range(K)]
        + [
            pl.BlockSpec((BR, D), lambda i: (i, 0)),
            pl.BlockSpec((D, 1), lambda i: (0, 0)),
            pl.BlockSpec((1,), lambda i: (0,)),
        ],
        out_specs=pl.BlockSpec((BR, D), lambda i: (i, 0)),
        out_shape=jax.ShapeDtypeStruct((N, D), jnp.float32),
    )(h0, *ps, nd, proj_w, proj_b)


@jax.jit
def kernel(x, edge_index, W1, b1, W2, b2, proj_w, proj_b):
    h0 = _mlp(x, W1, b1, W2, b2)
    pad = jnp.full((EBR * EC - E,), N, jnp.int32)
    srcM = jnp.concatenate([edge_index[0], pad]).reshape(EBR, EC)
    dstM = jnp.concatenate([edge_index[1], pad]).reshape(EBR, EC)
    ns, nd, nc = _sc_degnorm(srcM, dstM)
    ps = []
    p = _sc_hop0(h0, srcM, dstM, ns)
    ps.append(p)
    for _ in range(K - 1):
        p = _sc_hop(p, srcM, dstM, nc)
        ps.append(p)
    return _gating(h0, ps, nd, proj_w, proj_b)
